# Initial kernel scaffold; baseline (speedup 1.0000x reference)
import jax
import jax.numpy as jnp
from jax.experimental import pallas as pl


def kernel(x, edge_index, edge_attr, ne_w1, ne_b1, ne_w2, ne_b2, ne_g, ne_bln, ee_w1, ee_b1, ee_w2, ee_b2, ee_g, ee_bln, em_w1, em_b1, em_w2, em_b2, em_g, em_bln, nm_w1, nm_b1, nm_w2, nm_b2, nm_g, nm_bln, dec_w1, dec_b1, dec_w2, dec_b2):
    raise NotImplementedError("write your pallas kernel here")


# speedup vs baseline: 1.9040x; 1.9040x over previous
import functools

import jax
import jax.numpy as jnp
from jax import lax
from jax.experimental import pallas as pl
from jax.experimental.pallas import tpu as pltpu
from jax.experimental.pallas import tpu_sc as plsc

N = 10000
E = 320000
DIN_N = 128
H = 64
OUT = 3
DEPTH = 4

NPAD = 10240
NBLK = 1280
EBLK = 3200

_NC = 2
_NS = 16
_NW = _NC * _NS
_EPW = E // _NW
_CH = 80
_NCH = _EPW // _CH
_RPT = NPAD // _NS

_f32 = jnp.float32


def _ln_rows(t, g, b):
    m = jnp.mean(t, axis=-1, keepdims=True)
    d = t - m
    v = jnp.mean(d * d, axis=-1, keepdims=True)
    return d * lax.rsqrt(v + 1e-5) * g + b


def _dot(a, b):
    return jnp.dot(a, b, preferred_element_type=_f32)


def _node_enc_body(x_ref, w1_ref, b1_ref, w2_ref, b2_ref, g_ref, bln_ref,
                   w1a_ref, w1b_ref, h_ref, td_ref, ts_ref):
    hmid = jnp.maximum(_dot(x_ref[...], w1_ref[...]) + b1_ref[...], 0.0)
    t = _dot(hmid, w2_ref[...]) + b2_ref[...]
    h = _ln_rows(t, g_ref[...], bln_ref[...])
    h_ref[...] = h
    td_ref[...] = _dot(h, w1a_ref[...])
    ts_ref[...] = _dot(h, w1b_ref[...])


def _node_enc(xp, w1, b1, w2, b2, g, bln, w1a, w1b):
    grid = (NPAD // NBLK,)
    row = lambda i: (i, 0)
    bcast = lambda i: (0, 0)
    return pl.pallas_call(
        _node_enc_body,
        grid=grid,
        in_specs=[
            pl.BlockSpec((NBLK, DIN_N), row),
            pl.BlockSpec((DIN_N, H), bcast),
            pl.BlockSpec((1, H), bcast),
            pl.BlockSpec((H, H), bcast),
            pl.BlockSpec((1, H), bcast),
            pl.BlockSpec((1, H), bcast),
            pl.BlockSpec((1, H), bcast),
            pl.BlockSpec((H, H), bcast),
            pl.BlockSpec((H, H), bcast),
        ],
        out_specs=[pl.BlockSpec((NBLK, H), row)] * 3,
        out_shape=[jax.ShapeDtypeStruct((NPAD, H), _f32)] * 3,
    )(xp, w1, b1, w2, b2, g, bln, w1a, w1b)


def _edge_enc_body(a_ref, w1_ref, b1_ref, w2_ref, b2_ref, g_ref, bln_ref,
                   ea_ref):
    a = a_ref[...]
    h1 = b1_ref[...]
    for k in range(4):
        h1 = h1 + a[:, k:k + 1] * w1_ref[k:k + 1, :]
    hmid = jnp.maximum(h1, 0.0)
    t = _dot(hmid, w2_ref[...]) + b2_ref[...]
    ea_ref[...] = _ln_rows(t, g_ref[...], bln_ref[...])


def _edge_enc(attr, w1, b1, w2, b2, g, bln):
    grid = (E // EBLK,)
    row = lambda i: (i, 0)
    bcast = lambda i: (0, 0)
    return pl.pallas_call(
        _edge_enc_body,
        grid=grid,
        in_specs=[
            pl.BlockSpec((EBLK, 4), row),
            pl.BlockSpec((4, H), bcast),
            pl.BlockSpec((1, H), bcast),
            pl.BlockSpec((H, H), bcast),
            pl.BlockSpec((1, H), bcast),
            pl.BlockSpec((1, H), bcast),
            pl.BlockSpec((1, H), bcast),
        ],
        out_specs=pl.BlockSpec((EBLK, H), row),
        out_shape=jax.ShapeDtypeStruct((E, H), _f32),
    )(attr, w1, b1, w2, b2, g, bln)


def _edge_msg_body(gd_ref, gs_ref, ea_ref, w1c_ref, b1_ref, w2_ref, b2_ref,
                   g_ref, bln_ref, ue_ref):
    ea = ea_ref[...]
    pre = gd_ref[...] + gs_ref[...] + _dot(ea, w1c_ref[...]) + b1_ref[...]
    hmid = jnp.maximum(pre, 0.0)
    t = _dot(hmid, w2_ref[...]) + b2_ref[...]
    ue_ref[...] = _ln_rows(t, g_ref[...], bln_ref[...]) + ea


def _edge_msg(gd, gs, ea, w1c, b1, w2, b2, g, bln):
    grid = (E // EBLK,)
    row = lambda i: (i, 0)
    bcast = lambda i: (0, 0)
    return pl.pallas_call(
        _edge_msg_body,
        grid=grid,
        in_specs=[
            pl.BlockSpec((EBLK, H), row),
            pl.BlockSpec((EBLK, H), row),
            pl.BlockSpec((EBLK, H), row),
            pl.BlockSpec((H, H), bcast),
            pl.BlockSpec((1, H), bcast),
            pl.BlockSpec((H, H), bcast),
            pl.BlockSpec((1, H), bcast),
            pl.BlockSpec((1, H), bcast),
            pl.BlockSpec((1, H), bcast),
        ],
        out_specs=pl.BlockSpec((EBLK, H), row),
        out_shape=jax.ShapeDtypeStruct((E, H), _f32),
    )(gd, gs, ea, w1c, b1, w2, b2, g, bln)


def _node_upd_body(h_ref, a0_ref, a1_ref, w1h_ref, w1a_ref, b1_ref, w2_ref,
                   b2_ref, g_ref, bln_ref, w1an_ref, w1bn_ref,
                   h_out, td_out, ts_out):
    h = h_ref[...]
    agg = a0_ref[...] + a1_ref[...]
    pre = _dot(h, w1h_ref[...]) + _dot(agg, w1a_ref[...]) + b1_ref[...]
    hmid = jnp.maximum(pre, 0.0)
    t = _dot(hmid, w2_ref[...]) + b2_ref[...]
    hn = h + _ln_rows(t, g_ref[...], bln_ref[...])
    h_out[...] = hn
    td_out[...] = _dot(hn, w1an_ref[...])
    ts_out[...] = _dot(hn, w1bn_ref[...])


def _node_upd(h, agg, w1h, w1a, b1, w2, b2, g, bln, w1an, w1bn):
    grid = (NPAD // NBLK,)
    row = lambda i: (i, 0)
    bcast = lambda i: (0, 0)
    return pl.pallas_call(
        _node_upd_body,
        grid=grid,
        in_specs=[
            pl.BlockSpec((NBLK, H), row),
            pl.BlockSpec((NBLK, H), row),
            pl.BlockSpec((NBLK, H), row),
            pl.BlockSpec((H, H), bcast),
            pl.BlockSpec((H, H), bcast),
            pl.BlockSpec((1, H), bcast),
            pl.BlockSpec((H, H), bcast),
            pl.BlockSpec((1, H), bcast),
            pl.BlockSpec((1, H), bcast),
            pl.BlockSpec((1, H), bcast),
            pl.BlockSpec((H, H), bcast),
            pl.BlockSpec((H, H), bcast),
        ],
        out_specs=[pl.BlockSpec((NBLK, H), row)] * 3,
        out_shape=[jax.ShapeDtypeStruct((NPAD, H), _f32)] * 3,
    )(h, agg[0], agg[1], w1h, w1a, b1, w2, b2, g, bln, w1an, w1bn)


def _node_upd_dec_body(h_ref, a0_ref, a1_ref, w1h_ref, w1a_ref, b1_ref,
                       w2_ref, b2_ref, g_ref, bln_ref, dw1_ref, db1_ref,
                       dw2_ref, db2_ref, o_ref):
    h = h_ref[...]
    agg = a0_ref[...] + a1_ref[...]
    pre = _dot(h, w1h_ref[...]) + _dot(agg, w1a_ref[...]) + b1_ref[...]
    hmid = jnp.maximum(pre, 0.0)
    t = _dot(hmid, w2_ref[...]) + b2_ref[...]
    hn = h + _ln_rows(t, g_ref[...], bln_ref[...])
    dmid = jnp.maximum(_dot(hn, dw1_ref[...]) + db1_ref[...], 0.0)
    o_ref[...] = _dot(dmid, dw2_ref[...]) + db2_ref[...]


def _node_upd_dec(h, agg, w1h, w1a, b1, w2, b2, g, bln, dw1, db1, dw2p, db2p):
    grid = (NPAD // NBLK,)
    row = lambda i: (i, 0)
    bcast = lambda i: (0, 0)
    return pl.pallas_call(
        _node_upd_dec_body,
        grid=grid,
        in_specs=[
            pl.BlockSpec((NBLK, H), row),
            pl.BlockSpec((NBLK, H), row),
            pl.BlockSpec((NBLK, H), row),
            pl.BlockSpec((H, H), bcast),
            pl.BlockSpec((H, H), bcast),
            pl.BlockSpec((1, H), bcast),
            pl.BlockSpec((H, H), bcast),
            pl.BlockSpec((1, H), bcast),
            pl.BlockSpec((1, H), bcast),
            pl.BlockSpec((1, H), bcast),
            pl.BlockSpec((H, H), bcast),
            pl.BlockSpec((1, H), bcast),
            pl.BlockSpec((H, 8), bcast),
            pl.BlockSpec((1, 8), bcast),
        ],
        out_specs=pl.BlockSpec((NBLK, 8), row),
        out_shape=jax.ShapeDtypeStruct((NPAD, 8), _f32),
    )(h, agg[0], agg[1], w1h, w1a, b1, w2, b2, g, bln, dw1, db1,
      dw2p, db2p)


@functools.lru_cache(maxsize=1)
def _sc_mesh():
    return plsc.VectorSubcoreMesh(
        core_axis_name="c", subcore_axis_name="s", num_cores=_NC,
        num_subcores=_NS)


def _gather_body(td_hbm, ts_hbm, di_hbm, si_hbm, gd_hbm, gs_hbm,
                 idx_d, idx_s, rows_d, rows_s, sem_d, sem_s):
    wid = lax.axis_index("s") * _NC + lax.axis_index("c")
    base_w = wid * _EPW

    @pl.loop(0, _NCH)
    def _chunk(ci):
        base = base_w + ci * _CH
        pltpu.sync_copy(di_hbm.at[pl.ds(base, _CH)], idx_d)
        pltpu.sync_copy(si_hbm.at[pl.ds(base, _CH)], idx_s)
        cd = pltpu.async_copy(td_hbm.at[idx_d], rows_d, sem_d)
        cs = pltpu.async_copy(ts_hbm.at[idx_s], rows_s, sem_s)
        cd.wait()
        cs.wait()
        pltpu.sync_copy(rows_d, gd_hbm.at[pl.ds(base, _CH)])
        pltpu.sync_copy(rows_s, gs_hbm.at[pl.ds(base, _CH)])


def _gather2(td, ts, dst_idx, src_idx):
    fn = pl.kernel(
        _gather_body,
        out_type=(jax.ShapeDtypeStruct((E, H), _f32),
                  jax.ShapeDtypeStruct((E, H), _f32)),
        mesh=_sc_mesh(),
        scratch_types=[
            pltpu.VMEM((_CH,), jnp.int32),
            pltpu.VMEM((_CH,), jnp.int32),
            pltpu.VMEM((_CH, H), _f32),
            pltpu.VMEM((_CH, H), _f32),
            pltpu.SemaphoreType.DMA,
            pltpu.SemaphoreType.DMA,
        ],
        compiler_params=pltpu.CompilerParams(use_tc_tiling_on_sc=False),
    )
    return fn(td, ts, dst_idx, src_idx)


def _scatter_body(ue_hbm, si_hbm, out_hbm, idx_v, rows_v, zero_v, acc):
    cid = lax.axis_index("c")
    sid = lax.axis_index("s")
    wid = sid * _NC + cid

    zvec = jnp.zeros((16,), _f32)

    @pl.loop(0, _CH * H // 16)
    def _z(i):
        zero_v[i // 4, pl.ds((i % 4) * 16, 16)] = zvec

    @pl.loop(0, _RPT // _CH)
    def _zs(j):
        pltpu.sync_copy(zero_v, acc.at[pl.ds(sid * _RPT + j * _CH, _CH)])

    plsc.subcore_barrier()

    @pl.loop(0, _NCH)
    def _chunk(ci):
        base = wid * _EPW + ci * _CH
        pltpu.sync_copy(si_hbm.at[pl.ds(base, _CH)], idx_v)
        pltpu.sync_copy(ue_hbm.at[pl.ds(base, _CH)], rows_v)
        pltpu.sync_copy(rows_v, acc.at[idx_v], add=True)

    plsc.subcore_barrier()

    @pl.loop(0, _RPT // _CH)
    def _dump(j):
        st = sid * _RPT + j * _CH
        pltpu.sync_copy(acc.at[pl.ds(st, _CH)], rows_v)
        pltpu.sync_copy(rows_v, out_hbm.at[cid, pl.ds(st, _CH)])


def _scatter_partials(ue, src_idx):
    fn = pl.kernel(
        _scatter_body,
        out_type=jax.ShapeDtypeStruct((_NC, NPAD, H), _f32),
        mesh=_sc_mesh(),
        scratch_types=[
            pltpu.VMEM((_CH,), jnp.int32),
            pltpu.VMEM((_CH, H), _f32),
            pltpu.VMEM((_CH, H), _f32),
            pltpu.VMEM_SHARED((NPAD, H), _f32),
        ],
        compiler_params=pltpu.CompilerParams(use_tc_tiling_on_sc=False),
    )
    return fn(ue, src_idx)


def kernel(x, edge_index, edge_attr, ne_w1, ne_b1, ne_w2, ne_b2, ne_g,
           ne_bln, ee_w1, ee_b1, ee_w2, ee_b2, ee_g, ee_bln, em_w1, em_b1,
           em_w2, em_b2, em_g, em_bln, nm_w1, nm_b1, nm_w2, nm_b2, nm_g,
           nm_bln, dec_w1, dec_b1, dec_w2, dec_b2):
    src = edge_index[0]
    dst = edge_index[1]
    xp = jnp.pad(x, ((0, NPAD - N), (0, 0)))

    r = lambda v: v.reshape(1, -1)

    h, td, ts = _node_enc(xp, ne_w1, r(ne_b1), ne_w2, r(ne_b2), r(ne_g),
                          r(ne_bln), em_w1[0, :H], em_w1[0, H:2 * H])
    ea = _edge_enc(edge_attr, ee_w1, r(ee_b1), ee_w2, r(ee_b2), r(ee_g),
                   r(ee_bln))

    dw2p = jnp.pad(dec_w2, ((0, 0), (0, 8 - OUT)))
    db2p = jnp.pad(dec_b2, ((0, 8 - OUT),)).reshape(1, 8)

    gd, gs = _gather2(td, ts, dst, src)
    ue = _edge_msg(gd, gs, ea, em_w1[0, 2 * H:], r(em_b1[0]), em_w2[0],
                   r(em_b2[0]), r(em_g[0]), r(em_bln[0]))
    out = None
    for i in range(DEPTH):
        gd, gs = _gather2(td, ts, dst, src)
        ue = _edge_msg(gd, gs, ea, em_w1[i, 2 * H:], r(em_b1[i]), em_w2[i],
                       r(em_b2[i]), r(em_g[i]), r(em_bln[i]))
        agg = _scatter_partials(ue, src)
        if i < DEPTH - 1:
            h, td, ts = _node_upd(h, agg, nm_w1[i, :H], nm_w1[i, H:],
                                  r(nm_b1[i]), nm_w2[i], r(nm_b2[i]),
                                  r(nm_g[i]), r(nm_bln[i]),
                                  em_w1[i + 1, :H], em_w1[i + 1, H:2 * H])
        else:
            out = _node_upd_dec(h, agg, nm_w1[i, :H], nm_w1[i, H:],
                                r(nm_b1[i]), nm_w2[i], r(nm_b2[i]),
                                r(nm_g[i]), r(nm_bln[i]),
                                dec_w1, r(dec_b1), dw2p, db2p)
        ea = ue
    return out[:N, :OUT]


# speedup vs baseline: 3.0777x; 1.6164x over previous
import functools

import jax
import jax.numpy as jnp
from jax import lax
from jax.experimental import pallas as pl
from jax.experimental.pallas import tpu as pltpu
from jax.experimental.pallas import tpu_sc as plsc

N = 10000
E = 320000
DIN_N = 128
H = 64
OUT = 3
DEPTH = 4

NPAD = 10240
NBLK = 1280
EBLK = 3200
E2 = E // 2
EBLK2 = EBLK // 2

_NC = 2
_NS = 16
_NW = _NC * _NS
_EPW = E // _NW
_CH = 80
_NCH = _EPW // _CH
_RPT = NPAD // _NS

_f32 = jnp.float32


def _ln_rows(t, g, b):
    m = jnp.mean(t, axis=-1, keepdims=True)
    d = t - m
    v = jnp.mean(d * d, axis=-1, keepdims=True)
    return d * lax.rsqrt(v + 1e-5) * g + b


def _dot(a, b):
    return jnp.dot(a, b, preferred_element_type=_f32)


def _node_enc_body(x_ref, w1_ref, b1_ref, w2_ref, b2_ref, g_ref, bln_ref,
                   w1a_ref, w1b_ref, h_ref, td_ref, ts_ref):
    hmid = jnp.maximum(_dot(x_ref[...], w1_ref[...]) + b1_ref[...], 0.0)
    t = _dot(hmid, w2_ref[...]) + b2_ref[...]
    h = _ln_rows(t, g_ref[...], bln_ref[...])
    h_ref[...] = h
    td_ref[...] = _dot(h, w1a_ref[...])
    ts_ref[...] = _dot(h, w1b_ref[...])


def _node_enc(xp, w1, b1, w2, b2, g, bln, w1a, w1b):
    grid = (NPAD // NBLK,)
    row = lambda i: (i, 0)
    bcast = lambda i: (0, 0)
    return pl.pallas_call(
        _node_enc_body,
        grid=grid,
        in_specs=[
            pl.BlockSpec((NBLK, DIN_N), row),
            pl.BlockSpec((DIN_N, H), bcast),
            pl.BlockSpec((1, H), bcast),
            pl.BlockSpec((H, H), bcast),
            pl.BlockSpec((1, H), bcast),
            pl.BlockSpec((1, H), bcast),
            pl.BlockSpec((1, H), bcast),
            pl.BlockSpec((H, H), bcast),
            pl.BlockSpec((H, H), bcast),
        ],
        out_specs=[pl.BlockSpec((NBLK, H), row)] * 3,
        out_shape=[jax.ShapeDtypeStruct((NPAD, H), _f32)] * 3,
    )(xp, w1, b1, w2, b2, g, bln, w1a, w1b)


def _ln_packed(t, g, b, mb):
    m = _dot(t, mb)
    d = t - m
    v = _dot(d * d, mb)
    return d * lax.rsqrt(v + 1e-5) * g + b


def _edge_enc_body(a_ref, w1_ref, b1_ref, w2_ref, b2_ref, g_ref, bln_ref,
                   mb_ref, ea_ref):
    hmid = jnp.maximum(_dot(a_ref[...], w1_ref[...]) + b1_ref[...], 0.0)
    t = _dot(hmid, w2_ref[...]) + b2_ref[...]
    ea_ref[...] = _ln_packed(t, g_ref[...], bln_ref[...], mb_ref[...])


def _edge_enc(attr2, w1p, b1, w2, b2, g, bln, mb):
    grid = (E2 // EBLK2,)
    row = lambda i: (i, 0)
    bcast = lambda i: (0, 0)
    return pl.pallas_call(
        _edge_enc_body,
        grid=grid,
        in_specs=[
            pl.BlockSpec((EBLK2, 8), row),
            pl.BlockSpec((8, 2 * H), bcast),
            pl.BlockSpec((1, 2 * H), bcast),
            pl.BlockSpec((2 * H, 2 * H), bcast),
            pl.BlockSpec((1, 2 * H), bcast),
            pl.BlockSpec((1, 2 * H), bcast),
            pl.BlockSpec((1, 2 * H), bcast),
            pl.BlockSpec((2 * H, 2 * H), bcast),
        ],
        out_specs=pl.BlockSpec((EBLK2, 2 * H), row),
        out_shape=jax.ShapeDtypeStruct((E2, 2 * H), _f32),
    )(attr2, w1p, b1, w2, b2, g, bln, mb)


def _edge_msg_body(gd_ref, gs_ref, ea_ref, w1c_ref, b1_ref, w2_ref, b2_ref,
                   g_ref, bln_ref, mb_ref, ue_ref):
    ea = ea_ref[...]
    pre = gd_ref[...] + gs_ref[...] + _dot(ea, w1c_ref[...]) + b1_ref[...]
    hmid = jnp.maximum(pre, 0.0)
    t = _dot(hmid, w2_ref[...]) + b2_ref[...]
    ue_ref[...] = _ln_packed(t, g_ref[...], bln_ref[...], mb_ref[...]) + ea


def _edge_msg(gd, gs, ea, w1c, b1, w2, b2, g, bln, mb):
    grid = (E2 // EBLK2,)
    row = lambda i: (i, 0)
    bcast = lambda i: (0, 0)
    w = lambda: pl.BlockSpec((2 * H, 2 * H), bcast)
    v = lambda: pl.BlockSpec((1, 2 * H), bcast)
    return pl.pallas_call(
        _edge_msg_body,
        grid=grid,
        in_specs=[
            pl.BlockSpec((EBLK2, 2 * H), row),
            pl.BlockSpec((EBLK2, 2 * H), row),
            pl.BlockSpec((EBLK2, 2 * H), row),
            w(), v(), w(), v(), v(), v(), w(),
        ],
        out_specs=pl.BlockSpec((EBLK2, 2 * H), row),
        out_shape=jax.ShapeDtypeStruct((E2, 2 * H), _f32),
    )(gd, gs, ea, w1c, b1, w2, b2, g, bln, mb)


def _node_upd_body(h_ref, a0_ref, a1_ref, w1h_ref, w1a_ref, b1_ref, w2_ref,
                   b2_ref, g_ref, bln_ref, w1an_ref, w1bn_ref,
                   h_out, td_out, ts_out):
    h = h_ref[...]
    agg = a0_ref[...] + a1_ref[...]
    pre = _dot(h, w1h_ref[...]) + _dot(agg, w1a_ref[...]) + b1_ref[...]
    hmid = jnp.maximum(pre, 0.0)
    t = _dot(hmid, w2_ref[...]) + b2_ref[...]
    hn = h + _ln_rows(t, g_ref[...], bln_ref[...])
    h_out[...] = hn
    td_out[...] = _dot(hn, w1an_ref[...])
    ts_out[...] = _dot(hn, w1bn_ref[...])


def _node_upd(h, agg, w1h, w1a, b1, w2, b2, g, bln, w1an, w1bn):
    grid = (NPAD // NBLK,)
    row = lambda i: (i, 0)
    bcast = lambda i: (0, 0)
    return pl.pallas_call(
        _node_upd_body,
        grid=grid,
        in_specs=[
            pl.BlockSpec((NBLK, H), row),
            pl.BlockSpec((NBLK, H), row),
            pl.BlockSpec((NBLK, H), row),
            pl.BlockSpec((H, H), bcast),
            pl.BlockSpec((H, H), bcast),
            pl.BlockSpec((1, H), bcast),
            pl.BlockSpec((H, H), bcast),
            pl.BlockSpec((1, H), bcast),
            pl.BlockSpec((1, H), bcast),
            pl.BlockSpec((1, H), bcast),
            pl.BlockSpec((H, H), bcast),
            pl.BlockSpec((H, H), bcast),
        ],
        out_specs=[pl.BlockSpec((NBLK, H), row)] * 3,
        out_shape=[jax.ShapeDtypeStruct((NPAD, H), _f32)] * 3,
    )(h, agg[0], agg[1], w1h, w1a, b1, w2, b2, g, bln, w1an, w1bn)


def _node_upd_dec_body(h_ref, a0_ref, a1_ref, w1h_ref, w1a_ref, b1_ref,
                       w2_ref, b2_ref, g_ref, bln_ref, dw1_ref, db1_ref,
                       dw2_ref, db2_ref, o_ref):
    h = h_ref[...]
    agg = a0_ref[...] + a1_ref[...]
    pre = _dot(h, w1h_ref[...]) + _dot(agg, w1a_ref[...]) + b1_ref[...]
    hmid = jnp.maximum(pre, 0.0)
    t = _dot(hmid, w2_ref[...]) + b2_ref[...]
    hn = h + _ln_rows(t, g_ref[...], bln_ref[...])
    dmid = jnp.maximum(_dot(hn, dw1_ref[...]) + db1_ref[...], 0.0)
    o_ref[...] = _dot(dmid, dw2_ref[...]) + db2_ref[...]


def _node_upd_dec(h, agg, w1h, w1a, b1, w2, b2, g, bln, dw1, db1, dw2p, db2p):
    grid = (NPAD // NBLK,)
    row = lambda i: (i, 0)
    bcast = lambda i: (0, 0)
    return pl.pallas_call(
        _node_upd_dec_body,
        grid=grid,
        in_specs=[
            pl.BlockSpec((NBLK, H), row),
            pl.BlockSpec((NBLK, H), row),
            pl.BlockSpec((NBLK, H), row),
            pl.BlockSpec((H, H), bcast),
            pl.BlockSpec((H, H), bcast),
            pl.BlockSpec((1, H), bcast),
            pl.BlockSpec((H, H), bcast),
            pl.BlockSpec((1, H), bcast),
            pl.BlockSpec((1, H), bcast),
            pl.BlockSpec((1, H), bcast),
            pl.BlockSpec((H, H), bcast),
            pl.BlockSpec((1, H), bcast),
            pl.BlockSpec((H, 8), bcast),
            pl.BlockSpec((1, 8), bcast),
        ],
        out_specs=pl.BlockSpec((NBLK, 8), row),
        out_shape=jax.ShapeDtypeStruct((NPAD, 8), _f32),
    )(h, agg[0], agg[1], w1h, w1a, b1, w2, b2, g, bln, dw1, db1,
      dw2p, db2p)


@functools.lru_cache(maxsize=1)
def _sc_mesh():
    return plsc.VectorSubcoreMesh(
        core_axis_name="c", subcore_axis_name="s", num_cores=_NC,
        num_subcores=_NS)


def _gather_body(td_hbm, ts_hbm, di_hbm, si_hbm, gd_hbm, gs_hbm,
                 idx_d, idx_s, rows_d, rows_s, sem_d, sem_s):
    wid = lax.axis_index("s") * _NC + lax.axis_index("c")
    base_w = wid * _EPW

    @pl.loop(0, _NCH)
    def _chunk(ci):
        base = base_w + ci * _CH
        pltpu.sync_copy(di_hbm.at[pl.ds(base, _CH)], idx_d)
        pltpu.sync_copy(si_hbm.at[pl.ds(base, _CH)], idx_s)
        cd = pltpu.async_copy(td_hbm.at[idx_d], rows_d, sem_d)
        cs = pltpu.async_copy(ts_hbm.at[idx_s], rows_s, sem_s)
        cd.wait()
        cs.wait()
        pltpu.sync_copy(rows_d, gd_hbm.at[pl.ds(base, _CH)])
        pltpu.sync_copy(rows_s, gs_hbm.at[pl.ds(base, _CH)])


def _gather2(td, ts, dst_idx, src_idx):
    fn = pl.kernel(
        _gather_body,
        out_type=(jax.ShapeDtypeStruct((E, H), _f32),
                  jax.ShapeDtypeStruct((E, H), _f32)),
        mesh=_sc_mesh(),
        scratch_types=[
            pltpu.VMEM((_CH,), jnp.int32),
            pltpu.VMEM((_CH,), jnp.int32),
            pltpu.VMEM((_CH, H), _f32),
            pltpu.VMEM((_CH, H), _f32),
            pltpu.SemaphoreType.DMA,
            pltpu.SemaphoreType.DMA,
        ],
        compiler_params=pltpu.CompilerParams(use_tc_tiling_on_sc=False),
    )
    return fn(td, ts, dst_idx, src_idx)


def _scatter_body(ue_hbm, si_hbm, out_hbm, idx_v, rows_v, zero_v, acc):
    cid = lax.axis_index("c")
    sid = lax.axis_index("s")
    wid = sid * _NC + cid

    zvec = jnp.zeros((16,), _f32)

    @pl.loop(0, _CH * H // 16)
    def _z(i):
        zero_v[i // 4, pl.ds((i % 4) * 16, 16)] = zvec

    @pl.loop(0, _RPT // _CH)
    def _zs(j):
        pltpu.sync_copy(zero_v, acc.at[pl.ds(sid * _RPT + j * _CH, _CH)])

    plsc.subcore_barrier()

    @pl.loop(0, _NCH)
    def _chunk(ci):
        base = wid * _EPW + ci * _CH
        pltpu.sync_copy(si_hbm.at[pl.ds(base, _CH)], idx_v)
        pltpu.sync_copy(ue_hbm.at[pl.ds(base, _CH)], rows_v)
        pltpu.sync_copy(rows_v, acc.at[idx_v], add=True)

    plsc.subcore_barrier()

    @pl.loop(0, _RPT // _CH)
    def _dump(j):
        st = sid * _RPT + j * _CH
        pltpu.sync_copy(acc.at[pl.ds(st, _CH)], rows_v)
        pltpu.sync_copy(rows_v, out_hbm.at[cid, pl.ds(st, _CH)])


def _scatter_partials(ue, src_idx):
    fn = pl.kernel(
        _scatter_body,
        out_type=jax.ShapeDtypeStruct((_NC, NPAD, H), _f32),
        mesh=_sc_mesh(),
        scratch_types=[
            pltpu.VMEM((_CH,), jnp.int32),
            pltpu.VMEM((_CH, H), _f32),
            pltpu.VMEM((_CH, H), _f32),
            pltpu.VMEM_SHARED((NPAD, H), _f32),
        ],
        compiler_params=pltpu.CompilerParams(use_tc_tiling_on_sc=False),
    )
    return fn(ue, src_idx)


def kernel(x, edge_index, edge_attr, ne_w1, ne_b1, ne_w2, ne_b2, ne_g,
           ne_bln, ee_w1, ee_b1, ee_w2, ee_b2, ee_g, ee_bln, em_w1, em_b1,
           em_w2, em_b2, em_g, em_bln, nm_w1, nm_b1, nm_w2, nm_b2, nm_g,
           nm_bln, dec_w1, dec_b1, dec_w2, dec_b2):
    src = edge_index[0]
    dst = edge_index[1]
    xp = jnp.pad(x, ((0, NPAD - N), (0, 0)))

    r = lambda v: v.reshape(1, -1)
    tile2 = lambda v: jnp.concatenate([v, v]).reshape(1, 2 * H)

    def bd(w):
        z = jnp.zeros_like(w)
        return jnp.concatenate([jnp.concatenate([w, z], axis=1),
                                jnp.concatenate([z, w], axis=1)], axis=0)

    mb = bd(jnp.full((H, H), 1.0 / H, _f32))

    h, td, ts = _node_enc(xp, ne_w1, r(ne_b1), ne_w2, r(ne_b2), r(ne_g),
                          r(ne_bln), em_w1[0, :H], em_w1[0, H:2 * H])

    attr2 = edge_attr.reshape(E2, 8)
    ee_w1p = jnp.concatenate(
        [jnp.concatenate([ee_w1, jnp.zeros_like(ee_w1)], axis=1),
         jnp.concatenate([jnp.zeros_like(ee_w1), ee_w1], axis=1)], axis=0)
    ea = _edge_enc(attr2, ee_w1p, tile2(ee_b1), bd(ee_w2), tile2(ee_b2),
                   tile2(ee_g), tile2(ee_bln), mb)

    dw2p = jnp.pad(dec_w2, ((0, 0), (0, 8 - OUT)))
    db2p = jnp.pad(dec_b2, ((0, 8 - OUT),)).reshape(1, 8)

    out = None
    for i in range(DEPTH):
        gd, gs = _gather2(td, ts, dst, src)
        ue = _edge_msg(gd.reshape(E2, 2 * H), gs.reshape(E2, 2 * H), ea,
                       bd(em_w1[i, 2 * H:]), tile2(em_b1[i]), bd(em_w2[i]),
                       tile2(em_b2[i]), tile2(em_g[i]), tile2(em_bln[i]), mb)
        agg = _scatter_partials(ue.reshape(E, H), src)
        if i < DEPTH - 1:
            h, td, ts = _node_upd(h, agg, nm_w1[i, :H], nm_w1[i, H:],
                                  r(nm_b1[i]), nm_w2[i], r(nm_b2[i]),
                                  r(nm_g[i]), r(nm_bln[i]),
                                  em_w1[i + 1, :H], em_w1[i + 1, H:2 * H])
        else:
            out = _node_upd_dec(h, agg, nm_w1[i, :H], nm_w1[i, H:],
                                r(nm_b1[i]), nm_w2[i], r(nm_b2[i]),
                                r(nm_g[i]), r(nm_bln[i]),
                                dec_w1, r(dec_b1), dw2p, db2p)
        ea = ue
    return out[:N, :OUT]


# speedup vs baseline: 4.1372x; 1.3442x over previous
import functools

import jax
import jax.numpy as jnp
from jax import lax
from jax.experimental import pallas as pl
from jax.experimental.pallas import tpu as pltpu
from jax.experimental.pallas import tpu_sc as plsc

N = 10000
E = 320000
DIN_N = 128
H = 64
OUT = 3
DEPTH = 4

NPAD = 10240
NBLK = 1280
EBLK = 3200
E2 = E // 2
EBLK2 = EBLK // 2

_NC = 2
_NS = 16
_NW = _NC * _NS
_EPW = E // _NW
_CH = 80
_NCH = _EPW // _CH
_RPT = NPAD // _NS

_f32 = jnp.float32


def _ln_rows(t, g, b):
    m = jnp.mean(t, axis=-1, keepdims=True)
    d = t - m
    v = jnp.mean(d * d, axis=-1, keepdims=True)
    return d * lax.rsqrt(v + 1e-5) * g + b


def _dot(a, b):
    return jnp.dot(a, b, preferred_element_type=_f32)


def _node_enc_body(x_ref, w1_ref, b1_ref, w2_ref, b2_ref, g_ref, bln_ref,
                   w1a_ref, w1b_ref, h_ref, td_ref, ts_ref):
    hmid = jnp.maximum(_dot(x_ref[...], w1_ref[...]) + b1_ref[...], 0.0)
    t = _dot(hmid, w2_ref[...]) + b2_ref[...]
    h = _ln_rows(t, g_ref[...], bln_ref[...])
    h_ref[...] = h
    td_ref[...] = _dot(h, w1a_ref[...])
    ts_ref[...] = _dot(h, w1b_ref[...])


def _node_enc(xp, w1, b1, w2, b2, g, bln, w1a, w1b):
    grid = (NPAD // NBLK,)
    row = lambda i: (i, 0)
    bcast = lambda i: (0, 0)
    return pl.pallas_call(
        _node_enc_body,
        grid=grid,
        in_specs=[
            pl.BlockSpec((NBLK, DIN_N), row),
            pl.BlockSpec((DIN_N, H), bcast),
            pl.BlockSpec((1, H), bcast),
            pl.BlockSpec((H, H), bcast),
            pl.BlockSpec((1, H), bcast),
            pl.BlockSpec((1, H), bcast),
            pl.BlockSpec((1, H), bcast),
            pl.BlockSpec((H, H), bcast),
            pl.BlockSpec((H, H), bcast),
        ],
        out_specs=[pl.BlockSpec((NBLK, H), row)] * 3,
        out_shape=[jax.ShapeDtypeStruct((NPAD, H), _f32)] * 3,
    )(xp, w1, b1, w2, b2, g, bln, w1a, w1b)


def _ln_packed(t, g, b, mb):
    m = _dot(t, mb)
    d = t - m
    v = _dot(d * d, mb)
    return d * lax.rsqrt(v + 1e-5) * g + b


def _edge_enc_body(a_ref, w1_ref, b1_ref, w2_ref, b2_ref, g_ref, bln_ref,
                   mb_ref, ea_ref):
    hmid = jnp.maximum(_dot(a_ref[...], w1_ref[...]) + b1_ref[...], 0.0)
    t = _dot(hmid, w2_ref[...]) + b2_ref[...]
    ea_ref[...] = _ln_packed(t, g_ref[...], bln_ref[...], mb_ref[...])


def _edge_enc(attr2, w1p, b1, w2, b2, g, bln, mb):
    grid = (E2 // EBLK2,)
    row = lambda i: (i, 0)
    bcast = lambda i: (0, 0)
    return pl.pallas_call(
        _edge_enc_body,
        grid=grid,
        in_specs=[
            pl.BlockSpec((EBLK2, 8), row),
            pl.BlockSpec((8, 2 * H), bcast),
            pl.BlockSpec((1, 2 * H), bcast),
            pl.BlockSpec((2 * H, 2 * H), bcast),
            pl.BlockSpec((1, 2 * H), bcast),
            pl.BlockSpec((1, 2 * H), bcast),
            pl.BlockSpec((1, 2 * H), bcast),
            pl.BlockSpec((2 * H, 2 * H), bcast),
        ],
        out_specs=pl.BlockSpec((EBLK2, 2 * H), row),
        out_shape=jax.ShapeDtypeStruct((E2, 2 * H), _f32),
    )(attr2, w1p, b1, w2, b2, g, bln, mb)


def _edge_msg_body(gd_ref, gs_ref, ea_ref, w1c_ref, b1_ref, w2_ref, b2_ref,
                   g_ref, bln_ref, mb_ref, ue_ref):
    ea = ea_ref[...]
    pre = gd_ref[...] + gs_ref[...] + _dot(ea, w1c_ref[...]) + b1_ref[...]
    hmid = jnp.maximum(pre, 0.0)
    t = _dot(hmid, w2_ref[...]) + b2_ref[...]
    ue_ref[...] = _ln_packed(t, g_ref[...], bln_ref[...], mb_ref[...]) + ea


def _edge_msg(gd, gs, ea, w1c, b1, w2, b2, g, bln, mb):
    grid = (E2 // EBLK2,)
    row = lambda i: (i, 0)
    bcast = lambda i: (0, 0)
    w = lambda: pl.BlockSpec((2 * H, 2 * H), bcast)
    v = lambda: pl.BlockSpec((1, 2 * H), bcast)
    return pl.pallas_call(
        _edge_msg_body,
        grid=grid,
        in_specs=[
            pl.BlockSpec((EBLK2, 2 * H), row),
            pl.BlockSpec((EBLK2, 2 * H), row),
            pl.BlockSpec((EBLK2, 2 * H), row),
            w(), v(), w(), v(), v(), v(), w(),
        ],
        out_specs=pl.BlockSpec((EBLK2, 2 * H), row),
        out_shape=jax.ShapeDtypeStruct((E2, 2 * H), _f32),
    )(gd, gs, ea, w1c, b1, w2, b2, g, bln, mb)


def _node_upd_body(h_ref, a0_ref, a1_ref, w1h_ref, w1a_ref, b1_ref, w2_ref,
                   b2_ref, g_ref, bln_ref, w1an_ref, w1bn_ref,
                   h_out, td_out, ts_out):
    h = h_ref[...]
    agg = a0_ref[...] + a1_ref[...]
    pre = _dot(h, w1h_ref[...]) + _dot(agg, w1a_ref[...]) + b1_ref[...]
    hmid = jnp.maximum(pre, 0.0)
    t = _dot(hmid, w2_ref[...]) + b2_ref[...]
    hn = h + _ln_rows(t, g_ref[...], bln_ref[...])
    h_out[...] = hn
    td_out[...] = _dot(hn, w1an_ref[...])
    ts_out[...] = _dot(hn, w1bn_ref[...])


def _node_upd(h, agg, w1h, w1a, b1, w2, b2, g, bln, w1an, w1bn):
    grid = (NPAD // NBLK,)
    row = lambda i: (i, 0)
    bcast = lambda i: (0, 0)
    return pl.pallas_call(
        _node_upd_body,
        grid=grid,
        in_specs=[
            pl.BlockSpec((NBLK, H), row),
            pl.BlockSpec((NBLK, H), row),
            pl.BlockSpec((NBLK, H), row),
            pl.BlockSpec((H, H), bcast),
            pl.BlockSpec((H, H), bcast),
            pl.BlockSpec((1, H), bcast),
            pl.BlockSpec((H, H), bcast),
            pl.BlockSpec((1, H), bcast),
            pl.BlockSpec((1, H), bcast),
            pl.BlockSpec((1, H), bcast),
            pl.BlockSpec((H, H), bcast),
            pl.BlockSpec((H, H), bcast),
        ],
        out_specs=[pl.BlockSpec((NBLK, H), row)] * 3,
        out_shape=[jax.ShapeDtypeStruct((NPAD, H), _f32)] * 3,
    )(h, agg[0], agg[1], w1h, w1a, b1, w2, b2, g, bln, w1an, w1bn)


def _node_upd_dec_body(h_ref, a0_ref, a1_ref, w1h_ref, w1a_ref, b1_ref,
                       w2_ref, b2_ref, g_ref, bln_ref, dw1_ref, db1_ref,
                       dw2_ref, db2_ref, o_ref):
    h = h_ref[...]
    agg = a0_ref[...] + a1_ref[...]
    pre = _dot(h, w1h_ref[...]) + _dot(agg, w1a_ref[...]) + b1_ref[...]
    hmid = jnp.maximum(pre, 0.0)
    t = _dot(hmid, w2_ref[...]) + b2_ref[...]
    hn = h + _ln_rows(t, g_ref[...], bln_ref[...])
    dmid = jnp.maximum(_dot(hn, dw1_ref[...]) + db1_ref[...], 0.0)
    o_ref[...] = _dot(dmid, dw2_ref[...]) + db2_ref[...]


def _node_upd_dec(h, agg, w1h, w1a, b1, w2, b2, g, bln, dw1, db1, dw2p, db2p):
    grid = (NPAD // NBLK,)
    row = lambda i: (i, 0)
    bcast = lambda i: (0, 0)
    return pl.pallas_call(
        _node_upd_dec_body,
        grid=grid,
        in_specs=[
            pl.BlockSpec((NBLK, H), row),
            pl.BlockSpec((NBLK, H), row),
            pl.BlockSpec((NBLK, H), row),
            pl.BlockSpec((H, H), bcast),
            pl.BlockSpec((H, H), bcast),
            pl.BlockSpec((1, H), bcast),
            pl.BlockSpec((H, H), bcast),
            pl.BlockSpec((1, H), bcast),
            pl.BlockSpec((1, H), bcast),
            pl.BlockSpec((1, H), bcast),
            pl.BlockSpec((H, H), bcast),
            pl.BlockSpec((1, H), bcast),
            pl.BlockSpec((H, 8), bcast),
            pl.BlockSpec((1, 8), bcast),
        ],
        out_specs=pl.BlockSpec((NBLK, 8), row),
        out_shape=jax.ShapeDtypeStruct((NPAD, 8), _f32),
    )(h, agg[0], agg[1], w1h, w1a, b1, w2, b2, g, bln, dw1, db1,
      dw2p, db2p)


@functools.lru_cache(maxsize=1)
def _sc_mesh():
    return plsc.VectorSubcoreMesh(
        core_axis_name="c", subcore_axis_name="s", num_cores=_NC,
        num_subcores=_NS)


def _gather_body(td_hbm, ts_hbm, di_hbm, si_hbm, gd_hbm, gs_hbm,
                 idx_d, idx_s, rows_d, rows_s,
                 sgd, sgs, swd, sws):
    wid = lax.axis_index("s") * _NC + lax.axis_index("c")
    base_w = wid * _EPW

    pltpu.sync_copy(di_hbm.at[pl.ds(base_w, _EPW)], idx_d)
    pltpu.sync_copy(si_hbm.at[pl.ds(base_w, _EPW)], idx_s)

    def g_d(ci, b):
        return pltpu.make_async_copy(
            td_hbm.at[idx_d.at[pl.ds(ci * _CH, _CH)]], rows_d.at[b],
            sgd.at[b])

    def g_s(ci, b):
        return pltpu.make_async_copy(
            ts_hbm.at[idx_s.at[pl.ds(ci * _CH, _CH)]], rows_s.at[b],
            sgs.at[b])

    def w_d(ci, b):
        return pltpu.make_async_copy(
            rows_d.at[b], gd_hbm.at[pl.ds(base_w + ci * _CH, _CH)],
            swd.at[b])

    def w_s(ci, b):
        return pltpu.make_async_copy(
            rows_s.at[b], gs_hbm.at[pl.ds(base_w + ci * _CH, _CH)],
            sws.at[b])

    g_d(0, 0).start()
    g_s(0, 0).start()

    @pl.loop(0, _NCH)
    def _chunk(ci):
        b = lax.rem(ci, 2)
        nb = 1 - b
        g_d(ci, b).wait()
        g_s(ci, b).wait()
        w_d(ci, b).start()
        w_s(ci, b).start()

        @pl.when(ci > 0)
        def _():
            w_d(ci - 1, nb).wait()
            w_s(ci - 1, nb).wait()

        @pl.when(ci + 1 < _NCH)
        def _():
            g_d(ci + 1, nb).start()
            g_s(ci + 1, nb).start()

    bl = (_NCH - 1) % 2
    w_d(_NCH - 1, bl).wait()
    w_s(_NCH - 1, bl).wait()


def _gather2(td, ts, dst_idx, src_idx):
    fn = pl.kernel(
        _gather_body,
        out_type=(jax.ShapeDtypeStruct((E, H), _f32),
                  jax.ShapeDtypeStruct((E, H), _f32)),
        mesh=_sc_mesh(),
        scratch_types=[
            pltpu.VMEM((_EPW,), jnp.int32),
            pltpu.VMEM((_EPW,), jnp.int32),
            pltpu.VMEM((2, _CH, H), _f32),
            pltpu.VMEM((2, _CH, H), _f32),
            pltpu.SemaphoreType.DMA((2,)),
            pltpu.SemaphoreType.DMA((2,)),
            pltpu.SemaphoreType.DMA((2,)),
            pltpu.SemaphoreType.DMA((2,)),
        ],
        compiler_params=pltpu.CompilerParams(use_tc_tiling_on_sc=False),
    )
    return fn(td, ts, dst_idx, src_idx)


def _scatter_body(ue_hbm, si_hbm, out_hbm, idx_v, rows_v, zero_v, acc,
                  seml, semc):
    cid = lax.axis_index("c")
    sid = lax.axis_index("s")
    wid = sid * _NC + cid
    base_w = wid * _EPW

    zvec = jnp.zeros((16,), _f32)

    @pl.loop(0, _CH * H // 16)
    def _z(i):
        zero_v[i // 4, pl.ds((i % 4) * 16, 16)] = zvec

    @pl.loop(0, _RPT // _CH)
    def _zs(j):
        pltpu.sync_copy(zero_v, acc.at[pl.ds(sid * _RPT + j * _CH, _CH)])

    pltpu.sync_copy(si_hbm.at[pl.ds(base_w, _EPW)], idx_v)

    plsc.subcore_barrier()

    def load(ci, b):
        return pltpu.make_async_copy(
            ue_hbm.at[pl.ds(base_w + ci * _CH, _CH)], rows_v.at[b],
            seml.at[b])

    def scat(ci, b):
        return pltpu.async_copy(
            rows_v.at[b], acc.at[idx_v.at[pl.ds(ci * _CH, _CH)]],
            semc.at[b], add=True)

    def scat_wait(ci, b):
        pltpu.make_async_copy(
            rows_v.at[b], acc.at[idx_v.at[pl.ds(ci * _CH, _CH)]],
            semc.at[b]).wait()

    load(0, 0).start()

    @pl.loop(0, _NCH)
    def _chunk(ci):
        b = lax.rem(ci, 2)
        nb = 1 - b
        load(ci, b).wait()
        scat(ci, b)

        @pl.when(ci > 0)
        def _():
            scat_wait(ci - 1, nb)

        @pl.when(ci + 1 < _NCH)
        def _():
            load(ci + 1, nb).start()

    scat_wait(_NCH - 1, (_NCH - 1) % 2)

    plsc.subcore_barrier()

    @pl.loop(0, _RPT // _CH)
    def _dump(j):
        st = sid * _RPT + j * _CH
        pltpu.sync_copy(acc.at[pl.ds(st, _CH)], rows_v.at[0])
        pltpu.sync_copy(rows_v.at[0], out_hbm.at[cid, pl.ds(st, _CH)])


def _scatter_partials(ue, src_idx):
    fn = pl.kernel(
        _scatter_body,
        out_type=jax.ShapeDtypeStruct((_NC, NPAD, H), _f32),
        mesh=_sc_mesh(),
        scratch_types=[
            pltpu.VMEM((_EPW,), jnp.int32),
            pltpu.VMEM((2, _CH, H), _f32),
            pltpu.VMEM((_CH, H), _f32),
            pltpu.VMEM_SHARED((NPAD, H), _f32),
            pltpu.SemaphoreType.DMA((2,)),
            pltpu.SemaphoreType.DMA((2,)),
        ],
        compiler_params=pltpu.CompilerParams(use_tc_tiling_on_sc=False),
    )
    return fn(ue, src_idx)


def kernel(x, edge_index, edge_attr, ne_w1, ne_b1, ne_w2, ne_b2, ne_g,
           ne_bln, ee_w1, ee_b1, ee_w2, ee_b2, ee_g, ee_bln, em_w1, em_b1,
           em_w2, em_b2, em_g, em_bln, nm_w1, nm_b1, nm_w2, nm_b2, nm_g,
           nm_bln, dec_w1, dec_b1, dec_w2, dec_b2):
    src = edge_index[0]
    dst = edge_index[1]
    xp = jnp.pad(x, ((0, NPAD - N), (0, 0)))

    r = lambda v: v.reshape(1, -1)
    tile2 = lambda v: jnp.concatenate([v, v]).reshape(1, 2 * H)

    def bd(w):
        z = jnp.zeros_like(w)
        return jnp.concatenate([jnp.concatenate([w, z], axis=1),
                                jnp.concatenate([z, w], axis=1)], axis=0)

    mb = bd(jnp.full((H, H), 1.0 / H, _f32))

    h, td, ts = _node_enc(xp, ne_w1, r(ne_b1), ne_w2, r(ne_b2), r(ne_g),
                          r(ne_bln), em_w1[0, :H], em_w1[0, H:2 * H])

    attr2 = edge_attr.reshape(E2, 8)
    ee_w1p = jnp.concatenate(
        [jnp.concatenate([ee_w1, jnp.zeros_like(ee_w1)], axis=1),
         jnp.concatenate([jnp.zeros_like(ee_w1), ee_w1], axis=1)], axis=0)
    ea = _edge_enc(attr2, ee_w1p, tile2(ee_b1), bd(ee_w2), tile2(ee_b2),
                   tile2(ee_g), tile2(ee_bln), mb)

    dw2p = jnp.pad(dec_w2, ((0, 0), (0, 8 - OUT)))
    db2p = jnp.pad(dec_b2, ((0, 8 - OUT),)).reshape(1, 8)

    out = None
    for i in range(DEPTH):
        gd, gs = _gather2(td, ts, dst, src)
        ue = _edge_msg(gd.reshape(E2, 2 * H), gs.reshape(E2, 2 * H), ea,
                       bd(em_w1[i, 2 * H:]), tile2(em_b1[i]), bd(em_w2[i]),
                       tile2(em_b2[i]), tile2(em_g[i]), tile2(em_bln[i]), mb)
        agg = _scatter_partials(ue.reshape(E, H), src)
        if i < DEPTH - 1:
            h, td, ts = _node_upd(h, agg, nm_w1[i, :H], nm_w1[i, H:],
                                  r(nm_b1[i]), nm_w2[i], r(nm_b2[i]),
                                  r(nm_g[i]), r(nm_bln[i]),
                                  em_w1[i + 1, :H], em_w1[i + 1, H:2 * H])
        else:
            out = _node_upd_dec(h, agg, nm_w1[i, :H], nm_w1[i, H:],
                                r(nm_b1[i]), nm_w2[i], r(nm_b2[i]),
                                r(nm_g[i]), r(nm_bln[i]),
                                dec_w1, r(dec_b1), dw2p, db2p)
        ea = ue
    return out[:N, :OUT]


# speedup vs baseline: 4.1494x; 1.0029x over previous
import functools

import jax
import jax.numpy as jnp
from jax import lax
from jax.experimental import pallas as pl
from jax.experimental.pallas import tpu as pltpu
from jax.experimental.pallas import tpu_sc as plsc

N = 10000
E = 320000
DIN_N = 128
H = 64
OUT = 3
DEPTH = 4

NPAD = 10240
NBLK = 1280
EBLK = 3200
E2 = E // 2
EBLK2 = EBLK // 2

_NC = 2
_NS = 16
_NW = _NC * _NS
_EPW = E // _NW
_CH = 80
_NCH = _EPW // _CH
_RPT = NPAD // _NS

_f32 = jnp.float32


def _ln_rows(t, g, b):
    m = jnp.mean(t, axis=-1, keepdims=True)
    d = t - m
    v = jnp.mean(d * d, axis=-1, keepdims=True)
    return d * lax.rsqrt(v + 1e-5) * g + b


def _dot(a, b):
    return jnp.dot(a, b, preferred_element_type=_f32)


def _node_enc_body(x_ref, w1_ref, b1_ref, w2_ref, b2_ref, g_ref, bln_ref,
                   w1a_ref, w1b_ref, h_ref, td_ref, ts_ref):
    hmid = jnp.maximum(_dot(x_ref[...], w1_ref[...]) + b1_ref[...], 0.0)
    t = _dot(hmid, w2_ref[...]) + b2_ref[...]
    h = _ln_rows(t, g_ref[...], bln_ref[...])
    h_ref[...] = h
    td_ref[...] = _dot(h, w1a_ref[...])
    ts_ref[...] = _dot(h, w1b_ref[...])


def _node_enc(xp, w1, b1, w2, b2, g, bln, w1a, w1b):
    grid = (NPAD // NBLK,)
    row = lambda i: (i, 0)
    bcast = lambda i: (0, 0)
    return pl.pallas_call(
        _node_enc_body,
        grid=grid,
        in_specs=[
            pl.BlockSpec((NBLK, DIN_N), row),
            pl.BlockSpec((DIN_N, H), bcast),
            pl.BlockSpec((1, H), bcast),
            pl.BlockSpec((H, H), bcast),
            pl.BlockSpec((1, H), bcast),
            pl.BlockSpec((1, H), bcast),
            pl.BlockSpec((1, H), bcast),
            pl.BlockSpec((H, H), bcast),
            pl.BlockSpec((H, H), bcast),
        ],
        out_specs=[pl.BlockSpec((NBLK, H), row)] * 3,
        out_shape=[jax.ShapeDtypeStruct((NPAD, H), _f32)] * 3,
    )(xp, w1, b1, w2, b2, g, bln, w1a, w1b)


def _ln_packed(t, g, b, mb):
    m = _dot(t, mb)
    d = t - m
    v = _dot(d * d, mb)
    return d * lax.rsqrt(v + 1e-5) * g + b


def _edge_enc1_body(a_ref, w_ref, b_ref, o_ref):
    o_ref[...] = jnp.maximum(_dot(a_ref[...], w_ref[...]) + b_ref[...], 0.0)


def _edge_enc1(attr32, w1k, b1t):
    E32 = E // 32
    BLK32 = 400
    grid = (E32 // BLK32,)
    return pl.pallas_call(
        _edge_enc1_body,
        grid=grid,
        in_specs=[
            pl.BlockSpec((BLK32, 128), lambda i: (i, 0)),
            pl.BlockSpec((128, 2048), lambda i: (0, 0)),
            pl.BlockSpec((1, 2048), lambda i: (0, 0)),
        ],
        out_specs=pl.BlockSpec((BLK32, 2048), lambda i: (i, 0)),
        out_shape=jax.ShapeDtypeStruct((E32, 2048), _f32),
    )(attr32, w1k, b1t)


def _edge_msg_first_body(gd_ref, gs_ref, h1_ref, ew2_ref, eb2_ref, eg_ref,
                         ebln_ref, w1c_ref, b1_ref, w2_ref, b2_ref, g_ref,
                         bln_ref, mb_ref, ue_ref):
    mb = mb_ref[...]
    te = _dot(h1_ref[...], ew2_ref[...]) + eb2_ref[...]
    ea = _ln_packed(te, eg_ref[...], ebln_ref[...], mb)
    pre = gd_ref[...] + gs_ref[...] + _dot(ea, w1c_ref[...]) + b1_ref[...]
    hmid = jnp.maximum(pre, 0.0)
    t = _dot(hmid, w2_ref[...]) + b2_ref[...]
    ue_ref[...] = _ln_packed(t, g_ref[...], bln_ref[...], mb) + ea


def _edge_msg_first(gd, gs, h1, ew2, eb2, eg, ebln, w1c, b1, w2, b2, g, bln,
                    mb):
    grid = (E2 // EBLK2,)
    row = lambda i: (i, 0)
    bcast = lambda i: (0, 0)
    w = lambda: pl.BlockSpec((2 * H, 2 * H), bcast)
    v = lambda: pl.BlockSpec((1, 2 * H), bcast)
    return pl.pallas_call(
        _edge_msg_first_body,
        grid=grid,
        in_specs=[
            pl.BlockSpec((EBLK2, 2 * H), row),
            pl.BlockSpec((EBLK2, 2 * H), row),
            pl.BlockSpec((EBLK2, 2 * H), row),
            w(), v(), v(), v(), w(), v(), w(), v(), v(), v(), w(),
        ],
        out_specs=pl.BlockSpec((EBLK2, 2 * H), row),
        out_shape=jax.ShapeDtypeStruct((E2, 2 * H), _f32),
    )(gd, gs, h1, ew2, eb2, eg, ebln, w1c, b1, w2, b2, g, bln, mb)


def _edge_msg_body(gd_ref, gs_ref, ea_ref, w1c_ref, b1_ref, w2_ref, b2_ref,
                   g_ref, bln_ref, mb_ref, ue_ref):
    ea = ea_ref[...]
    pre = gd_ref[...] + gs_ref[...] + _dot(ea, w1c_ref[...]) + b1_ref[...]
    hmid = jnp.maximum(pre, 0.0)
    t = _dot(hmid, w2_ref[...]) + b2_ref[...]
    ue_ref[...] = _ln_packed(t, g_ref[...], bln_ref[...], mb_ref[...]) + ea


def _edge_msg(gd, gs, ea, w1c, b1, w2, b2, g, bln, mb):
    grid = (E2 // EBLK2,)
    row = lambda i: (i, 0)
    bcast = lambda i: (0, 0)
    w = lambda: pl.BlockSpec((2 * H, 2 * H), bcast)
    v = lambda: pl.BlockSpec((1, 2 * H), bcast)
    return pl.pallas_call(
        _edge_msg_body,
        grid=grid,
        in_specs=[
            pl.BlockSpec((EBLK2, 2 * H), row),
            pl.BlockSpec((EBLK2, 2 * H), row),
            pl.BlockSpec((EBLK2, 2 * H), row),
            w(), v(), w(), v(), v(), v(), w(),
        ],
        out_specs=pl.BlockSpec((EBLK2, 2 * H), row),
        out_shape=jax.ShapeDtypeStruct((E2, 2 * H), _f32),
    )(gd, gs, ea, w1c, b1, w2, b2, g, bln, mb)


def _node_upd_body(h_ref, a0_ref, a1_ref, w1h_ref, w1a_ref, b1_ref, w2_ref,
                   b2_ref, g_ref, bln_ref, w1an_ref, w1bn_ref,
                   h_out, td_out, ts_out):
    h = h_ref[...]
    agg = a0_ref[...] + a1_ref[...]
    pre = _dot(h, w1h_ref[...]) + _dot(agg, w1a_ref[...]) + b1_ref[...]
    hmid = jnp.maximum(pre, 0.0)
    t = _dot(hmid, w2_ref[...]) + b2_ref[...]
    hn = h + _ln_rows(t, g_ref[...], bln_ref[...])
    h_out[...] = hn
    td_out[...] = _dot(hn, w1an_ref[...])
    ts_out[...] = _dot(hn, w1bn_ref[...])


def _node_upd(h, agg0, agg1, w1h, w1a, b1, w2, b2, g, bln, w1an, w1bn):
    grid = (NPAD // NBLK,)
    row = lambda i: (i, 0)
    bcast = lambda i: (0, 0)
    return pl.pallas_call(
        _node_upd_body,
        grid=grid,
        in_specs=[
            pl.BlockSpec((NBLK, H), row),
            pl.BlockSpec((NBLK, H), row),
            pl.BlockSpec((NBLK, H), row),
            pl.BlockSpec((H, H), bcast),
            pl.BlockSpec((H, H), bcast),
            pl.BlockSpec((1, H), bcast),
            pl.BlockSpec((H, H), bcast),
            pl.BlockSpec((1, H), bcast),
            pl.BlockSpec((1, H), bcast),
            pl.BlockSpec((1, H), bcast),
            pl.BlockSpec((H, H), bcast),
            pl.BlockSpec((H, H), bcast),
        ],
        out_specs=[pl.BlockSpec((NBLK, H), row)] * 3,
        out_shape=[jax.ShapeDtypeStruct((NPAD, H), _f32)] * 3,
    )(h, agg0, agg1, w1h, w1a, b1, w2, b2, g, bln, w1an, w1bn)


def _node_upd_dec_body(h_ref, a0_ref, a1_ref, w1h_ref, w1a_ref, b1_ref,
                       w2_ref, b2_ref, g_ref, bln_ref, dw1_ref, db1_ref,
                       dw2_ref, db2_ref, o_ref):
    h = h_ref[...]
    agg = a0_ref[...] + a1_ref[...]
    pre = _dot(h, w1h_ref[...]) + _dot(agg, w1a_ref[...]) + b1_ref[...]
    hmid = jnp.maximum(pre, 0.0)
    t = _dot(hmid, w2_ref[...]) + b2_ref[...]
    hn = h + _ln_rows(t, g_ref[...], bln_ref[...])
    dmid = jnp.maximum(_dot(hn, dw1_ref[...]) + db1_ref[...], 0.0)
    o_ref[...] = _dot(dmid, dw2_ref[...]) + db2_ref[...]


def _node_upd_dec(h, agg0, agg1, w1h, w1a, b1, w2, b2, g, bln, dw1, db1,
                  dw2p, db2p):
    grid = (NPAD // NBLK,)
    row = lambda i: (i, 0)
    bcast = lambda i: (0, 0)
    return pl.pallas_call(
        _node_upd_dec_body,
        grid=grid,
        in_specs=[
            pl.BlockSpec((NBLK, H), row),
            pl.BlockSpec((NBLK, H), row),
            pl.BlockSpec((NBLK, H), row),
            pl.BlockSpec((H, H), bcast),
            pl.BlockSpec((H, H), bcast),
            pl.BlockSpec((1, H), bcast),
            pl.BlockSpec((H, H), bcast),
            pl.BlockSpec((1, H), bcast),
            pl.BlockSpec((1, H), bcast),
            pl.BlockSpec((1, H), bcast),
            pl.BlockSpec((H, H), bcast),
            pl.BlockSpec((1, H), bcast),
            pl.BlockSpec((H, 8), bcast),
            pl.BlockSpec((1, 8), bcast),
        ],
        out_specs=pl.BlockSpec((NBLK, 8), row),
        out_shape=jax.ShapeDtypeStruct((NPAD, 8), _f32),
    )(h, agg0, agg1, w1h, w1a, b1, w2, b2, g, bln, dw1, db1,
      dw2p, db2p)


@functools.lru_cache(maxsize=1)
def _sc_mesh():
    return plsc.VectorSubcoreMesh(
        core_axis_name="c", subcore_axis_name="s", num_cores=_NC,
        num_subcores=_NS)


def _gather_body(td_hbm, ts_hbm, di_hbm, si_hbm, gd_hbm, gs_hbm,
                 idx_d, idx_s, rows_d, rows_s,
                 sgd, sgs, swd, sws):
    wid = lax.axis_index("s") * _NC + lax.axis_index("c")
    base_w = wid * _EPW

    pltpu.sync_copy(di_hbm.at[pl.ds(base_w, _EPW)], idx_d)
    pltpu.sync_copy(si_hbm.at[pl.ds(base_w, _EPW)], idx_s)

    def g_d(ci, b):
        return pltpu.make_async_copy(
            td_hbm.at[idx_d.at[pl.ds(ci * _CH, _CH)]], rows_d.at[b],
            sgd.at[b])

    def g_s(ci, b):
        return pltpu.make_async_copy(
            ts_hbm.at[idx_s.at[pl.ds(ci * _CH, _CH)]], rows_s.at[b],
            sgs.at[b])

    def w_d(ci, b):
        return pltpu.make_async_copy(
            rows_d.at[b], gd_hbm.at[pl.ds(base_w + ci * _CH, _CH)],
            swd.at[b])

    def w_s(ci, b):
        return pltpu.make_async_copy(
            rows_s.at[b], gs_hbm.at[pl.ds(base_w + ci * _CH, _CH)],
            sws.at[b])

    g_d(0, 0).start()
    g_s(0, 0).start()

    @pl.loop(0, _NCH)
    def _chunk(ci):
        b = lax.rem(ci, 2)
        nb = 1 - b
        g_d(ci, b).wait()
        g_s(ci, b).wait()
        w_d(ci, b).start()
        w_s(ci, b).start()

        @pl.when(ci > 0)
        def _():
            w_d(ci - 1, nb).wait()
            w_s(ci - 1, nb).wait()

        @pl.when(ci + 1 < _NCH)
        def _():
            g_d(ci + 1, nb).start()
            g_s(ci + 1, nb).start()

    bl = (_NCH - 1) % 2
    w_d(_NCH - 1, bl).wait()
    w_s(_NCH - 1, bl).wait()


def _gather2(td, ts, dst_idx, src_idx):
    fn = pl.kernel(
        _gather_body,
        out_type=(jax.ShapeDtypeStruct((E, H), _f32),
                  jax.ShapeDtypeStruct((E, H), _f32)),
        mesh=_sc_mesh(),
        scratch_types=[
            pltpu.VMEM((_EPW,), jnp.int32),
            pltpu.VMEM((_EPW,), jnp.int32),
            pltpu.VMEM((2, _CH, H), _f32),
            pltpu.VMEM((2, _CH, H), _f32),
            pltpu.SemaphoreType.DMA((2,)),
            pltpu.SemaphoreType.DMA((2,)),
            pltpu.SemaphoreType.DMA((2,)),
            pltpu.SemaphoreType.DMA((2,)),
        ],
        compiler_params=pltpu.CompilerParams(use_tc_tiling_on_sc=False),
    )
    return fn(td, ts, dst_idx, src_idx)


def _scatter_body(ue_hbm, si_hbm, out0_hbm, out1_hbm, idx_v, rows_v, zero_v,
                  acc, seml, semc):
    cid = lax.axis_index("c")
    sid = lax.axis_index("s")
    wid = sid * _NC + cid
    base_w = wid * _EPW

    zvec = jnp.zeros((16,), _f32)

    @pl.loop(0, _CH * H // 16)
    def _z(i):
        zero_v[i // 4, pl.ds((i % 4) * 16, 16)] = zvec

    @pl.loop(0, _RPT // _CH)
    def _zs(j):
        pltpu.sync_copy(zero_v, acc.at[pl.ds(sid * _RPT + j * _CH, _CH)])

    pltpu.sync_copy(si_hbm.at[pl.ds(base_w, _EPW)], idx_v)

    plsc.subcore_barrier()

    def load(ci, b):
        return pltpu.make_async_copy(
            ue_hbm.at[pl.ds(base_w + ci * _CH, _CH)], rows_v.at[b],
            seml.at[b])

    def scat(ci, b):
        return pltpu.async_copy(
            rows_v.at[b], acc.at[idx_v.at[pl.ds(ci * _CH, _CH)]],
            semc.at[b], add=True)

    def scat_wait(ci, b):
        pltpu.make_async_copy(
            rows_v.at[b], acc.at[idx_v.at[pl.ds(ci * _CH, _CH)]],
            semc.at[b]).wait()

    load(0, 0).start()

    @pl.loop(0, _NCH)
    def _chunk(ci):
        b = lax.rem(ci, 2)
        nb = 1 - b
        load(ci, b).wait()
        scat(ci, b)

        @pl.when(ci > 0)
        def _():
            scat_wait(ci - 1, nb)

        @pl.when(ci + 1 < _NCH)
        def _():
            load(ci + 1, nb).start()

    scat_wait(_NCH - 1, (_NCH - 1) % 2)

    plsc.subcore_barrier()

    @pl.loop(0, _RPT // _CH)
    def _dump(j):
        st = sid * _RPT + j * _CH
        pltpu.sync_copy(acc.at[pl.ds(st, _CH)], rows_v.at[0])

        @pl.when(cid == 0)
        def _():
            pltpu.sync_copy(rows_v.at[0], out0_hbm.at[pl.ds(st, _CH)])

        @pl.when(cid == 1)
        def _():
            pltpu.sync_copy(rows_v.at[0], out1_hbm.at[pl.ds(st, _CH)])


def _scatter_partials(ue, src_idx):
    fn = pl.kernel(
        _scatter_body,
        out_type=(jax.ShapeDtypeStruct((NPAD, H), _f32),
                  jax.ShapeDtypeStruct((NPAD, H), _f32)),
        mesh=_sc_mesh(),
        scratch_types=[
            pltpu.VMEM((_EPW,), jnp.int32),
            pltpu.VMEM((2, _CH, H), _f32),
            pltpu.VMEM((_CH, H), _f32),
            pltpu.VMEM_SHARED((NPAD, H), _f32),
            pltpu.SemaphoreType.DMA((2,)),
            pltpu.SemaphoreType.DMA((2,)),
        ],
        compiler_params=pltpu.CompilerParams(use_tc_tiling_on_sc=False),
    )
    return fn(ue, src_idx)


def kernel(x, edge_index, edge_attr, ne_w1, ne_b1, ne_w2, ne_b2, ne_g,
           ne_bln, ee_w1, ee_b1, ee_w2, ee_b2, ee_g, ee_bln, em_w1, em_b1,
           em_w2, em_b2, em_g, em_bln, nm_w1, nm_b1, nm_w2, nm_b2, nm_g,
           nm_bln, dec_w1, dec_b1, dec_w2, dec_b2):
    src = edge_index[0]
    dst = edge_index[1]
    xp = jnp.pad(x, ((0, NPAD - N), (0, 0)))

    r = lambda v: v.reshape(1, -1)
    tile2 = lambda v: jnp.concatenate([v, v]).reshape(1, 2 * H)

    def bd(w):
        z = jnp.zeros_like(w)
        return jnp.concatenate([jnp.concatenate([w, z], axis=1),
                                jnp.concatenate([z, w], axis=1)], axis=0)

    mb = bd(jnp.full((H, H), 1.0 / H, _f32))

    h, td, ts = _node_enc(xp, ne_w1, r(ne_b1), ne_w2, r(ne_b2), r(ne_g),
                          r(ne_bln), em_w1[0, :H], em_w1[0, H:2 * H])

    attr32 = edge_attr.reshape(E // 32, 128)
    ee_w1k = jnp.kron(jnp.eye(32, dtype=_f32), ee_w1)
    ee_b1t = jnp.tile(ee_b1, 32).reshape(1, 2048)
    h1 = _edge_enc1(attr32, ee_w1k, ee_b1t).reshape(E2, 2 * H)

    dw2p = jnp.pad(dec_w2, ((0, 0), (0, 8 - OUT)))
    db2p = jnp.pad(dec_b2, ((0, 8 - OUT),)).reshape(1, 8)

    out = None
    ea = None
    for i in range(DEPTH):
        gd, gs = _gather2(td, ts, dst, src)
        gdp = gd.reshape(E2, 2 * H)
        gsp = gs.reshape(E2, 2 * H)
        if i == 0:
            ue = _edge_msg_first(gdp, gsp, h1, bd(ee_w2), tile2(ee_b2),
                                 tile2(ee_g), tile2(ee_bln),
                                 bd(em_w1[0, 2 * H:]), tile2(em_b1[0]),
                                 bd(em_w2[0]), tile2(em_b2[0]),
                                 tile2(em_g[0]), tile2(em_bln[0]), mb)
        else:
            ue = _edge_msg(gdp, gsp, ea, bd(em_w1[i, 2 * H:]),
                           tile2(em_b1[i]), bd(em_w2[i]), tile2(em_b2[i]),
                           tile2(em_g[i]), tile2(em_bln[i]), mb)
        agg0, agg1 = _scatter_partials(ue.reshape(E, H), src)
        if i < DEPTH - 1:
            h, td, ts = _node_upd(h, agg0, agg1, nm_w1[i, :H], nm_w1[i, H:],
                                  r(nm_b1[i]), nm_w2[i], r(nm_b2[i]),
                                  r(nm_g[i]), r(nm_bln[i]),
                                  em_w1[i + 1, :H], em_w1[i + 1, H:2 * H])
        else:
            out = _node_upd_dec(h, agg0, agg1, nm_w1[i, :H], nm_w1[i, H:],
                                r(nm_b1[i]), nm_w2[i], r(nm_b2[i]),
                                r(nm_g[i]), r(nm_bln[i]),
                                dec_w1, r(dec_b1), dw2p, db2p)
        ea = ue
    return out[:N, :OUT]
